# baseline (speedup 1.0000x reference)
import functools

import jax
import jax.numpy as jnp
from jax import lax
from jax.experimental import pallas as pl
from jax.experimental.pallas import tpu as pltpu

GROUPS = 4
K = 512
DG = 64
CTX = 7
PAD = 128


def _vq_kernel(x_ref, emb_ref, w_ref, out_ref, *, tt: int):
    t0 = pl.program_id(1) * tt
    for g in range(GROUPS):
        rows = slice(g * DG, (g + 1) * DG)
        prevtail = x_ref[0, rows, pl.ds(t0, PAD)]
        cur = x_ref[0, rows, pl.ds(t0 + PAD, tt)]
        buf = jnp.concatenate([prevtail, cur], axis=1)
        pred = jnp.zeros((DG, tt), jnp.float32)
        for s in range(1, CTX + 1):
            xs = pltpu.roll(buf, s, axis=1)[:, PAD:]
            w = w_ref[g, :, :, CTX - s]
            pred = pred + jnp.dot(w, xs, preferred_element_type=jnp.float32)
        r = cur - pred
        emb = emb_ref[g]
        er = jnp.dot(emb, r, preferred_element_type=jnp.float32)
        half_e2 = 0.5 * jnp.sum(emb * emb, axis=1)[:, None]
        score = er - half_e2
        m = jnp.max(score, axis=0, keepdims=True)
        kiota = lax.broadcasted_iota(jnp.int32, (K, tt), 0)
        first = jnp.min(jnp.where(score >= m, kiota, K), axis=0, keepdims=True)
        hard = (kiota == first).astype(jnp.float32)
        q = lax.dot_general(emb, hard, (((0,), (0,)), ((), ())),
                            preferred_element_type=jnp.float32)
        out_ref[0, rows, :] = pred + q


def _run(x, embedding, conv_w, tt: int):
    b, c, t = x.shape
    xp = jnp.pad(x, ((0, 0), (0, 0), (PAD, 0)))
    grid = (b, t // tt)
    return pl.pallas_call(
        functools.partial(_vq_kernel, tt=tt),
        grid=grid,
        in_specs=[
            pl.BlockSpec((1, c, t + PAD), lambda i, j: (i, 0, 0)),
            pl.BlockSpec((GROUPS, K, DG), lambda i, j: (0, 0, 0)),
            pl.BlockSpec((GROUPS, DG, DG, CTX), lambda i, j: (0, 0, 0, 0)),
        ],
        out_specs=pl.BlockSpec((1, c, tt), lambda i, j: (i, 0, j)),
        out_shape=jax.ShapeDtypeStruct((b, c, t), jnp.float32),
        compiler_params=pltpu.CompilerParams(
            dimension_semantics=("parallel", "arbitrary"),
        ),
    )(xp, embedding, conv_w)


def kernel(x, embedding, conv_w):
    return _run(x, embedding, conv_w, tt=512)


# speedup vs baseline: 4.7440x; 4.7440x over previous
import functools

import jax
import jax.numpy as jnp
from jax import lax
from jax.experimental import pallas as pl
from jax.experimental.pallas import tpu as pltpu

GROUPS = 4
K = 512
DG = 64
CTX = 7
PAD = 128


def _vq_kernel(x_ref, emb_ref, w2_ref, out_ref, *, tt: int):
    t0 = pl.program_id(1) * tt
    for p in range(GROUPS // 2):
        rows = slice(p * 2 * DG, (p + 1) * 2 * DG)
        prevtail = x_ref[0, rows, pl.ds(t0, PAD)]
        cur = x_ref[0, rows, pl.ds(t0 + PAD, tt)]
        cur_t = jnp.swapaxes(cur, 0, 1)
        tail_t = jnp.swapaxes(prevtail, 0, 1)[PAD - 8:, :]
        buf_t = jnp.concatenate([tail_t, cur_t], axis=0)
        pred_t = jnp.zeros((tt, 2 * DG), jnp.float32)
        for s in range(1, CTX + 1):
            xs = pltpu.roll(buf_t, s, axis=0)[8:, :]
            w = w2_ref[CTX - s, p]
            pred_t = pred_t + lax.dot_general(
                xs, w, (((1,), (1,)), ((), ())),
                preferred_element_type=jnp.float32)
        r_t = cur_t - pred_t
        pred = jnp.swapaxes(pred_t, 0, 1)
        for h in range(2):
            g = 2 * p + h
            rg_t = r_t[:, h * DG:(h + 1) * DG]
            emb = emb_ref[g]
            er = lax.dot_general(emb, rg_t, (((1,), (1,)), ((), ())),
                                 preferred_element_type=jnp.float32)
            half_e2 = 0.5 * jnp.sum(emb * emb, axis=1)[:, None]
            score = er - half_e2
            m = jnp.max(score, axis=0, keepdims=True)
            kiota = lax.broadcasted_iota(jnp.int32, (K, tt), 0)
            first = jnp.min(jnp.where(score >= m, kiota, K), axis=0,
                            keepdims=True)
            hard = (kiota == first).astype(jnp.float32)
            q = lax.dot_general(emb, hard, (((0,), (0,)), ((), ())),
                                preferred_element_type=jnp.float32)
            out_ref[0, g * DG:(g + 1) * DG, :] = pred[h * DG:(h + 1) * DG, :] + q


def _run(x, embedding, conv_w, tt: int):
    b, c, t = x.shape
    xp = jnp.pad(x, ((0, 0), (0, 0), (PAD, 0)))
    z = jnp.zeros((CTX, GROUPS // 2, DG, DG), jnp.float32)
    wt = jnp.transpose(conv_w, (3, 0, 1, 2))
    top = jnp.concatenate([wt[:, 0::2], z], axis=-1)
    bot = jnp.concatenate([z, wt[:, 1::2]], axis=-1)
    w2 = jnp.concatenate([top, bot], axis=-2)
    grid = (b, t // tt)
    return pl.pallas_call(
        functools.partial(_vq_kernel, tt=tt),
        grid=grid,
        in_specs=[
            pl.BlockSpec((1, c, t + PAD), lambda i, j: (i, 0, 0)),
            pl.BlockSpec((GROUPS, K, DG), lambda i, j: (0, 0, 0)),
            pl.BlockSpec((CTX, GROUPS // 2, 2 * DG, 2 * DG),
                         lambda i, j: (0, 0, 0, 0)),
        ],
        out_specs=pl.BlockSpec((1, c, tt), lambda i, j: (i, 0, j)),
        out_shape=jax.ShapeDtypeStruct((b, c, t), jnp.float32),
        compiler_params=pltpu.CompilerParams(
            dimension_semantics=("parallel", "arbitrary"),
        ),
    )(xp, embedding, w2)


def kernel(x, embedding, conv_w):
    return _run(x, embedding, conv_w, tt=512)


# speedup vs baseline: 7.0318x; 1.4823x over previous
import functools

import jax
import jax.numpy as jnp
from jax import lax
from jax.experimental import pallas as pl
from jax.experimental.pallas import tpu as pltpu

GROUPS = 4
K = 512
DG = 64
CTX = 7
PAD = 128


def _vq_kernel(x_ref, emb_ref, w2_ref, out_ref, *, tt: int):
    t0 = pl.program_id(1) * tt
    for p in range(GROUPS // 2):
        rows = slice(p * 2 * DG, (p + 1) * 2 * DG)
        prevtail = x_ref[0, rows, pl.ds(t0, PAD)]
        cur = x_ref[0, rows, pl.ds(t0 + PAD, tt)]
        cur_t = jnp.swapaxes(cur, 0, 1)
        tail_t = jnp.swapaxes(prevtail, 0, 1)[PAD - 8:, :]
        buf_t = jnp.concatenate([tail_t, cur_t], axis=0)
        pred_t = jnp.zeros((tt, 2 * DG), jnp.float32)
        for s in range(1, CTX + 1):
            xs = pltpu.roll(buf_t, s, axis=0)[8:, :]
            w = w2_ref[CTX - s, p]
            pred_t = pred_t + lax.dot_general(
                xs, w, (((1,), (1,)), ((), ())),
                preferred_element_type=jnp.float32)
        r_t = cur_t - pred_t
        pred = jnp.swapaxes(pred_t, 0, 1)
        for h in range(2):
            g = 2 * p + h
            rg_t = r_t[:, h * DG:(h + 1) * DG]
            emb = emb_ref[g]
            er = lax.dot_general(emb, rg_t, (((1,), (1,)), ((), ())),
                                 preferred_element_type=jnp.float32)
            half_e2 = 0.5 * jnp.sum(emb * emb, axis=1)[:, None]
            score = er - half_e2
            m = jnp.max(score, axis=0, keepdims=True)
            hard = (score == m).astype(jnp.float32)
            q = lax.dot_general(emb, hard,
                                (((0,), (0,)), ((), ())),
                                preferred_element_type=jnp.float32)
            out_ref[0, g * DG:(g + 1) * DG, :] = pred[h * DG:(h + 1) * DG, :] + q


def _run(x, embedding, conv_w, tt: int):
    b, c, t = x.shape
    xp = jnp.pad(x, ((0, 0), (0, 0), (PAD, 0)))
    z = jnp.zeros((CTX, GROUPS // 2, DG, DG), jnp.float32)
    wt = jnp.transpose(conv_w, (3, 0, 1, 2))
    top = jnp.concatenate([wt[:, 0::2], z], axis=-1)
    bot = jnp.concatenate([z, wt[:, 1::2]], axis=-1)
    w2 = jnp.concatenate([top, bot], axis=-2)
    grid = (b, t // tt)
    return pl.pallas_call(
        functools.partial(_vq_kernel, tt=tt),
        grid=grid,
        in_specs=[
            pl.BlockSpec((1, c, t + PAD), lambda i, j: (i, 0, 0)),
            pl.BlockSpec((GROUPS, K, DG), lambda i, j: (0, 0, 0)),
            pl.BlockSpec((CTX, GROUPS // 2, 2 * DG, 2 * DG),
                         lambda i, j: (0, 0, 0, 0)),
        ],
        out_specs=pl.BlockSpec((1, c, tt), lambda i, j: (i, 0, j)),
        out_shape=jax.ShapeDtypeStruct((b, c, t), jnp.float32),
        compiler_params=pltpu.CompilerParams(
            dimension_semantics=("parallel", "arbitrary"),
        ),
    )(xp, embedding, w2)


def kernel(x, embedding, conv_w):
    return _run(x, embedding, conv_w, tt=2048)


# speedup vs baseline: 8.1296x; 1.1561x over previous
import jax
import jax.numpy as jnp
from jax import lax
from jax.experimental import pallas as pl
from jax.experimental.pallas import tpu as pltpu

GROUPS = 4
K = 512
DG = 64
CTX = 7


def _vq_kernel(x_ref, emb_ref, w2_ref, out_ref):
    tt = x_ref.shape[2]
    for p in range(GROUPS // 2):
        rows = slice(p * 2 * DG, (p + 1) * 2 * DG)
        cur = x_ref[0, rows, :]
        cur_t = jnp.swapaxes(cur, 0, 1)
        buf_t = jnp.concatenate(
            [jnp.zeros((8, 2 * DG), jnp.float32), cur_t], axis=0)
        pred_t = jnp.zeros((tt, 2 * DG), jnp.float32)
        for s in range(1, CTX + 1):
            xs = pltpu.roll(buf_t, s, axis=0)[8:, :]
            w = w2_ref[CTX - s, p]
            pred_t = pred_t + lax.dot_general(
                xs, w, (((1,), (1,)), ((), ())),
                preferred_element_type=jnp.float32)
        r_t = cur_t - pred_t
        pred = jnp.swapaxes(pred_t, 0, 1)
        for h in range(2):
            g = 2 * p + h
            rg_t = r_t[:, h * DG:(h + 1) * DG]
            emb = emb_ref[g]
            er = lax.dot_general(emb, rg_t, (((1,), (1,)), ((), ())),
                                 preferred_element_type=jnp.float32)
            half_e2 = 0.5 * jnp.sum(emb * emb, axis=1)[:, None]
            score = er - half_e2
            m = jnp.max(score, axis=0, keepdims=True)
            hard = (score == m).astype(jnp.float32)
            q = lax.dot_general(emb, hard, (((0,), (0,)), ((), ())),
                                preferred_element_type=jnp.float32)
            out_ref[0, g * DG:(g + 1) * DG, :] = pred[h * DG:(h + 1) * DG, :] + q


def kernel(x, embedding, conv_w):
    b, c, t = x.shape
    z = jnp.zeros((CTX, GROUPS // 2, DG, DG), jnp.float32)
    wt = jnp.transpose(conv_w, (3, 0, 1, 2))
    top = jnp.concatenate([wt[:, 0::2], z], axis=-1)
    bot = jnp.concatenate([z, wt[:, 1::2]], axis=-1)
    w2 = jnp.concatenate([top, bot], axis=-2)
    return pl.pallas_call(
        _vq_kernel,
        grid=(b,),
        in_specs=[
            pl.BlockSpec((1, c, t), lambda i: (i, 0, 0)),
            pl.BlockSpec((GROUPS, K, DG), lambda i: (0, 0, 0)),
            pl.BlockSpec((CTX, GROUPS // 2, 2 * DG, 2 * DG),
                         lambda i: (0, 0, 0, 0)),
        ],
        out_specs=pl.BlockSpec((1, c, t), lambda i: (i, 0, 0)),
        out_shape=jax.ShapeDtypeStruct((b, c, t), jnp.float32),
        compiler_params=pltpu.CompilerParams(
            dimension_semantics=("arbitrary",),
        ),
    )(x, embedding, w2)
